# baseline (speedup 1.0000x reference)
import functools

import jax
import jax.numpy as jnp
from jax import lax
from jax.experimental import pallas as pl
from jax.experimental.pallas import tpu as pltpu
from jax.experimental.pallas import tpu_sc as plsc

_F = 128
_NC = 2
_NS = 16
_NW = _NC * _NS
_CHUNK = 64


@functools.lru_cache(maxsize=None)
def _make_kernel(P: int):
    assert P % _NW == 0
    ppw = P // _NW
    assert ppw % _CHUNK == 0
    nch = ppw // _CHUNK
    rows = 2 * _CHUNK

    mesh = plsc.VectorSubcoreMesh(
        core_axis_name="c", subcore_axis_name="s",
        num_cores=_NC, num_subcores=_NS)

    nbuf = 4
    assert nch % nbuf == 0

    @functools.partial(
        pl.kernel,
        out_type=jax.ShapeDtypeStruct((2 * P, _F), jnp.float32),
        mesh=mesh,
        scratch_types=[
            pltpu.VMEM((2 * ppw,), jnp.int32),
            pltpu.VMEM_SHARED((1024, _F), jnp.float32),
            [pltpu.VMEM((rows, _F), jnp.float32) for _ in range(nbuf)],
            [pltpu.SemaphoreType.DMA for _ in range(nbuf)],
            [pltpu.SemaphoreType.DMA for _ in range(nbuf)],
        ],
        compiler_params=pltpu.CompilerParams(needs_layout_passes=False),
    )
    def emb(idx_hbm, tab_hbm, out_hbm, idx, stab, bufs, gsems, osems):
        wid = lax.axis_index("s") * _NC + lax.axis_index("c")
        base = wid * ppw
        pltpu.sync_copy(idx_hbm.at[pl.ds(2 * base, 2 * ppw)], idx)

        @pl.when(lax.axis_index("s") == 0)
        def _():
            pltpu.sync_copy(tab_hbm, stab)

        plsc.subcore_barrier()

        def gather(ci, k):
            r0 = ci * rows
            pltpu.async_copy(
                stab.at[idx.at[pl.ds(r0, rows)]], bufs[k], gsems[k])

        def gather_wait(k):
            pltpu.make_async_copy(
                stab.at[idx.at[pl.ds(0, rows)]], bufs[k], gsems[k]).wait()

        def writeback(ci, k):
            r0 = 2 * base + ci * rows
            pltpu.async_copy(bufs[k], out_hbm.at[pl.ds(r0, rows)], osems[k])

        def writeback_wait(k):
            pltpu.make_async_copy(
                bufs[k], out_hbm.at[pl.ds(2 * base, rows)], osems[k]).wait()

        def stage(i, c0):
            ci = i * nbuf
            for k in range(nbuf):
                @pl.when(i > 0)
                def _(k=k):
                    writeback_wait(k)
                gather(ci + k, k)
            for k in range(nbuf):
                gather_wait(k)
                writeback(ci + k, k)
            return c0

        lax.fori_loop(0, nch // nbuf, stage, 0)
        for k in range(nbuf):
            writeback_wait(k)

    return emb


def kernel(x, col_embed, row_embed):
    b, n, _ = x.shape
    p = b * n
    tab = jnp.concatenate([col_embed, row_embed], axis=0)
    fused_idx = x.reshape(2 * p) + (jnp.arange(2 * p, dtype=jnp.int32) & 1) * 512
    out = _make_kernel(p)(fused_idx, tab)
    return out.reshape(b, n, 2, _F).swapaxes(2, 3)
